# baseline (speedup 1.0000x reference)
import functools

import jax
import jax.numpy as jnp
from jax import lax
from jax.experimental import pallas as pl
from jax.experimental.pallas import tpu as pltpu
from jax.experimental.pallas import tpu_sc as plsc

B = 1024
C = 100000
H = 128
K = 5
TEMP = 0.07

CBW = 12800
NCB = -(-C // CBW)
RB1 = 256
RB = 128

NEG_INF = float("-inf")
BIG_I32 = 2 ** 30

_NC, _NS = 2, 16
_NW = _NC * _NS


@functools.cache
def _sc_mesh():
    return plsc.VectorSubcoreMesh(core_axis_name="c", subcore_axis_name="s")


def _p1_body(logits_ref, labels_ref, vals_ref, idx_ref):
    cb = pl.program_id(1)
    x = logits_ref[...]
    lab = labels_ref[...]
    lcol = lax.broadcasted_iota(jnp.int32, (RB1, CBW), 1)
    gcol = cb * CBW + lcol
    ok = (gcol != lab) & (gcol < C)
    xm = jnp.where(ok, x, NEG_INF)
    vs, ids = [], []
    for _ in range(K):
        mx = jnp.max(xm, axis=1, keepdims=True)
        a = jnp.argmax(xm, axis=1).astype(jnp.int32)[:, None]
        vs.append(mx)
        ids.append(cb * CBW + a)
        xm = jnp.where(lcol == a, NEG_INF, xm)
    zf = jnp.full((RB1, 8 - K), NEG_INF, jnp.float32)
    zi = jnp.full((RB1, 8 - K), BIG_I32, jnp.int32)
    vals_ref[...] = jnp.concatenate(vs + [zf], axis=1).reshape(1, RB1, 8)
    idx_ref[...] = jnp.concatenate(ids + [zi], axis=1).reshape(1, RB1, 8)


def _block_topk(logits, labels2d):
    return pl.pallas_call(
        _p1_body,
        grid=(B // RB1, NCB),
        in_specs=[
            pl.BlockSpec((RB1, CBW), lambda rb, cb: (rb, cb)),
            pl.BlockSpec((RB1, 1), lambda rb, cb: (rb, 0)),
        ],
        out_specs=[
            pl.BlockSpec((1, RB1, 8), lambda rb, cb: (cb, rb, 0)),
            pl.BlockSpec((1, RB1, 8), lambda rb, cb: (cb, rb, 0)),
        ],
        out_shape=[
            jax.ShapeDtypeStruct((NCB, B, 8), jnp.float32),
            jax.ShapeDtypeStruct((NCB, B, 8), jnp.int32),
        ],
        compiler_params=pltpu.CompilerParams(
            dimension_semantics=("parallel", "arbitrary")),
    )(logits, labels2d)


def _p2_body(vals_ref, idx_ref, out_ref):
    v = vals_ref[...]
    gi = idx_ref[...]
    sels = []
    for _ in range(K):
        mx = jnp.max(v, axis=(0, 2), keepdims=True)
        sel = jnp.min(jnp.where(v == mx, gi, BIG_I32),
                      axis=(0, 2), keepdims=True)
        sels.append(sel[0])
        v = jnp.where(gi == sel, NEG_INF, v)
    neg = jnp.concatenate(sels, axis=1)
    out_ref[...] = jnp.concatenate(
        [neg, jnp.zeros((RB, 8 - K), jnp.int32)], axis=1)


def _merge_topk(vals, idx):
    return pl.pallas_call(
        _p2_body,
        grid=(B // RB,),
        in_specs=[
            pl.BlockSpec((NCB, RB, 8), lambda rb: (0, rb, 0)),
            pl.BlockSpec((NCB, RB, 8), lambda rb: (0, rb, 0)),
        ],
        out_specs=pl.BlockSpec((RB, 8), lambda rb: (rb, 0)),
        out_shape=jax.ShapeDtypeStruct((B, 8), jnp.int32),
    )(vals, idx)


def _sc_gather(table, idx, D):
    T = idx.shape[0]
    b_per_w = T // _NW
    npiece = -(-b_per_w // 128)
    piece = b_per_w // npiece
    idx3 = idx.reshape(_NW, npiece, piece)

    @functools.partial(
        pl.kernel, mesh=_sc_mesh(),
        compiler_params=pltpu.CompilerParams(use_tc_tiling_on_sc=False),
        out_type=jax.ShapeDtypeStruct((T, D), jnp.float32),
        scratch_types=[
            pltpu.VMEM((npiece, piece), jnp.int32),
            pltpu.VMEM((piece, D), jnp.float32),
            pltpu.SemaphoreType.DMA,
        ],
    )
    def k(table_hbm, idx_hbm, out_hbm, idx_v, rows_v, sem):
        wid = lax.axis_index("s") * _NC + lax.axis_index("c")
        base = wid * b_per_w
        pltpu.sync_copy(idx_hbm.at[wid], idx_v)
        for j in range(npiece):
            pltpu.async_copy(table_hbm.at[idx_v.at[j]], rows_v, sem).wait()
            pltpu.sync_copy(rows_v, out_hbm.at[pl.ds(base + j * piece, piece)])

    return k(table, idx3)


def _p4_body(spec_ref, w1t_ref, b1_ref, w2t_ref, b2_ref, g_ref, out_ref):
    x = spec_ref[...]
    h = jnp.maximum(
        jnp.dot(x, w1t_ref[...], preferred_element_type=jnp.float32)
        + b1_ref[...], 0.0)
    proj = (jnp.dot(h, w2t_ref[...], preferred_element_type=jnp.float32)
            + b2_ref[...])
    g = g_ref[...]
    pos = g[:B]
    negs = g[B:].reshape(B, K, H)

    def infonce(q, p):
        pos_sim = jnp.sum(q * p, axis=1, keepdims=True) / TEMP
        neg_sims = [jnp.sum(q * negs[:, k, :], axis=1, keepdims=True) / TEMP
                    for k in range(K)]
        logits = jnp.concatenate([pos_sim] + neg_sims, axis=1)
        mx = jnp.max(logits, axis=1, keepdims=True)
        lse = mx + jnp.log(jnp.sum(jnp.exp(logits - mx), axis=1, keepdims=True))
        return -jnp.mean(pos_sim - lse)

    loss = infonce(proj, pos) + infonce(pos, proj)
    out_ref[...] = jnp.full((1, 1), loss, jnp.float32)


def _final_loss(spec, w1t, b1, w2t, b2, g):
    return pl.pallas_call(
        _p4_body,
        out_shape=jax.ShapeDtypeStruct((1, 1), jnp.float32),
    )(spec, w1t, b1, w2t, b2, g)


def kernel(specialization_features, labels, teacher_logits, prototypes,
           W1, b1, W2, b2):
    labels_i = labels.astype(jnp.int32)
    labels2d = labels_i.reshape(B, 1)

    vals, idx = _block_topk(teacher_logits, labels2d)
    neg = _merge_topk(vals, idx)

    idx_all = jnp.concatenate([labels_i, neg[:, :K].reshape(B * K)])
    g = _sc_gather(prototypes, idx_all, H)

    loss = _final_loss(
        specialization_features,
        jnp.transpose(W1), b1.reshape(1, H),
        jnp.transpose(W2), b2.reshape(1, H),
        g,
    )
    return loss.reshape(())


# speedup vs baseline: 1.0944x; 1.0944x over previous
import functools

import jax
import jax.numpy as jnp
from jax import lax
from jax.experimental import pallas as pl
from jax.experimental.pallas import tpu as pltpu
from jax.experimental.pallas import tpu_sc as plsc

B = 1024
C = 100000
H = 128
K = 5
TEMP = 0.07

CBW = 12800
NCB = -(-C // CBW)
RB1 = 256
RB = 128

NEG_INF = float("-inf")
BIG_I32 = 2 ** 30

_NC, _NS = 2, 16
_NW = _NC * _NS


@functools.cache
def _sc_mesh():
    return plsc.VectorSubcoreMesh(core_axis_name="c", subcore_axis_name="s")


def _p1_body(logits_ref, labels_ref, vals_ref, idx_ref):
    cb = pl.program_id(1)
    x = logits_ref[...]
    lab = labels_ref[...]
    gcol = cb * CBW + lax.broadcasted_iota(jnp.int32, (RB1, CBW), 1)
    ok = (gcol != lab) & (gcol < C)
    xm = jnp.where(ok, x, NEG_INF)
    vs, ids = [], []
    for _ in range(K):
        mx = jnp.max(xm, axis=1, keepdims=True)
        sidx = jnp.min(jnp.where(xm == mx, gcol, BIG_I32),
                       axis=1, keepdims=True)
        vs.append(mx)
        ids.append(sidx)
        xm = jnp.where(gcol == sidx, NEG_INF, xm)
    zf = jnp.full((RB1, 8 - K), NEG_INF, jnp.float32)
    zi = jnp.full((RB1, 8 - K), BIG_I32, jnp.int32)
    vals_ref[...] = jnp.concatenate(vs + [zf], axis=1).reshape(1, RB1, 8)
    idx_ref[...] = jnp.concatenate(ids + [zi], axis=1).reshape(1, RB1, 8)


def _block_topk(logits, labels2d):
    return pl.pallas_call(
        _p1_body,
        grid=(B // RB1, NCB),
        in_specs=[
            pl.BlockSpec((RB1, CBW), lambda rb, cb: (rb, cb)),
            pl.BlockSpec((RB1, 1), lambda rb, cb: (rb, 0)),
        ],
        out_specs=[
            pl.BlockSpec((1, RB1, 8), lambda rb, cb: (cb, rb, 0)),
            pl.BlockSpec((1, RB1, 8), lambda rb, cb: (cb, rb, 0)),
        ],
        out_shape=[
            jax.ShapeDtypeStruct((NCB, B, 8), jnp.float32),
            jax.ShapeDtypeStruct((NCB, B, 8), jnp.int32),
        ],
        compiler_params=pltpu.CompilerParams(
            dimension_semantics=("parallel", "arbitrary")),
    )(logits, labels2d)


def _p2_body(vals_ref, idx_ref, out_ref):
    v = vals_ref[...]
    gi = idx_ref[...]
    sels = []
    for _ in range(K):
        mx = jnp.max(v, axis=(0, 2), keepdims=True)
        sel = jnp.min(jnp.where(v == mx, gi, BIG_I32),
                      axis=(0, 2), keepdims=True)
        sels.append(sel[0])
        v = jnp.where(gi == sel, NEG_INF, v)
    neg = jnp.concatenate(sels, axis=1)
    out_ref[...] = jnp.concatenate(
        [neg, jnp.zeros((RB, 8 - K), jnp.int32)], axis=1)


def _merge_topk(vals, idx):
    return pl.pallas_call(
        _p2_body,
        grid=(B // RB,),
        in_specs=[
            pl.BlockSpec((NCB, RB, 8), lambda rb: (0, rb, 0)),
            pl.BlockSpec((NCB, RB, 8), lambda rb: (0, rb, 0)),
        ],
        out_specs=pl.BlockSpec((RB, 8), lambda rb: (rb, 0)),
        out_shape=jax.ShapeDtypeStruct((B, 8), jnp.int32),
    )(vals, idx)


def _sc_gather(table, idx, D):
    T = idx.shape[0]
    b_per_w = T // _NW
    npiece = -(-b_per_w // 128)
    piece = b_per_w // npiece
    idx3 = idx.reshape(_NW, npiece, piece)

    @functools.partial(
        pl.kernel, mesh=_sc_mesh(),
        compiler_params=pltpu.CompilerParams(use_tc_tiling_on_sc=False),
        out_type=jax.ShapeDtypeStruct((T, D), jnp.float32),
        scratch_types=[
            pltpu.VMEM((npiece, piece), jnp.int32),
            pltpu.VMEM((piece, D), jnp.float32),
            pltpu.SemaphoreType.DMA,
        ],
    )
    def k(table_hbm, idx_hbm, out_hbm, idx_v, rows_v, sem):
        wid = lax.axis_index("s") * _NC + lax.axis_index("c")
        base = wid * b_per_w
        pltpu.sync_copy(idx_hbm.at[wid], idx_v)
        for j in range(npiece):
            pltpu.async_copy(table_hbm.at[idx_v.at[j]], rows_v, sem).wait()
            pltpu.sync_copy(rows_v, out_hbm.at[pl.ds(base + j * piece, piece)])

    return k(table, idx3)


def _p4_body(spec_ref, w1t_ref, b1_ref, w2t_ref, b2_ref, g_ref, out_ref):
    x = spec_ref[...]
    h = jnp.maximum(
        jnp.dot(x, w1t_ref[...], preferred_element_type=jnp.float32)
        + b1_ref[...], 0.0)
    proj = (jnp.dot(h, w2t_ref[...], preferred_element_type=jnp.float32)
            + b2_ref[...])
    g = g_ref[...]
    pos = g[:B]
    negs = g[B:].reshape(B, K, H)

    def infonce(q, p):
        pos_sim = jnp.sum(q * p, axis=1, keepdims=True) / TEMP
        neg_sims = [jnp.sum(q * negs[:, k, :], axis=1, keepdims=True) / TEMP
                    for k in range(K)]
        logits = jnp.concatenate([pos_sim] + neg_sims, axis=1)
        mx = jnp.max(logits, axis=1, keepdims=True)
        lse = mx + jnp.log(jnp.sum(jnp.exp(logits - mx), axis=1, keepdims=True))
        return -jnp.mean(pos_sim - lse)

    loss = infonce(proj, pos) + infonce(pos, proj)
    out_ref[...] = jnp.full((1, 1), loss, jnp.float32)


def _final_loss(spec, w1t, b1, w2t, b2, g):
    return pl.pallas_call(
        _p4_body,
        out_shape=jax.ShapeDtypeStruct((1, 1), jnp.float32),
    )(spec, w1t, b1, w2t, b2, g)


def kernel(specialization_features, labels, teacher_logits, prototypes,
           W1, b1, W2, b2):
    labels_i = labels.astype(jnp.int32)
    labels2d = labels_i.reshape(B, 1)

    vals, idx = _block_topk(teacher_logits, labels2d)
    neg = _merge_topk(vals, idx)

    idx_all = jnp.concatenate([labels_i, neg[:, :K].reshape(B * K)])
    g = _sc_gather(prototypes, idx_all, H)

    loss = _final_loss(
        specialization_features,
        jnp.transpose(W1), b1.reshape(1, H),
        jnp.transpose(W2), b2.reshape(1, H),
        g,
    )
    return loss.reshape(())
